# baseline (speedup 1.0000x reference)
import jax
import jax.numpy as jnp
from jax import lax
from jax.experimental import pallas as pl
from jax.experimental.pallas import tpu as pltpu
from jax.experimental.pallas import tpu_sc as plsc

B, C, H, W = 64, 768, 28, 28
HW = H * W
BC = B * C
NC, NS, L = 2, 16, 16
NW = NC * NS
BATCHES_PER_W = B // NW
ROWS_PER_W = BATCHES_PER_W * C
CHUNK = 64
NCHUNK = ROWS_PER_W // CHUNK


def _sc_shuffle(x_hbm, idx_hbm, out_hbm, perm_v, gidx_v, buf0, buf1,
                gsem0, gsem1, osem0, osem1):
    cid = lax.axis_index("c")
    sid = lax.axis_index("s")
    wid = sid * NC + cid
    b0 = wid * BATCHES_PER_W

    pltpu.sync_copy(idx_hbm, perm_v)
    for b in range(BATCHES_PER_W):
        base = (b0 + b) * C
        for j in range(C // L):
            gidx_v[pl.ds(b * C + j * L, L)] = perm_v[pl.ds(j * L, L)] + base

    bufs = (buf0, buf1)
    gsems = (gsem0, gsem1)
    osems = (osem0, osem1)
    out_base = b0 * C

    def gather(k):
        idx_slice = gidx_v.at[pl.ds(k * CHUNK, CHUNK)]
        return pltpu.async_copy(x_hbm.at[idx_slice], bufs[k % 2], gsems[k % 2])

    def scatter(k):
        dst = out_hbm.at[pl.ds(out_base + k * CHUNK, CHUNK)]
        return pltpu.async_copy(bufs[k % 2], dst, osems[k % 2])

    g_descs = [None] * NCHUNK
    o_descs = [None] * NCHUNK
    g_descs[0] = gather(0)
    for k in range(NCHUNK):
        g_descs[k].wait()
        o_descs[k] = scatter(k)
        if k + 1 < NCHUNK:
            if k >= 1:
                o_descs[k - 1].wait()
            g_descs[k + 1] = gather(k + 1)
    o_descs[NCHUNK - 2].wait()
    o_descs[NCHUNK - 1].wait()


@jax.jit
def _shuffle(x2, indices):
    mesh = plsc.VectorSubcoreMesh(core_axis_name="c", subcore_axis_name="s",
                                  num_cores=NC, num_subcores=NS)
    return pl.kernel(
        _sc_shuffle,
        out_type=jax.ShapeDtypeStruct((BC, HW), jnp.float32),
        mesh=mesh,
        scratch_types=[
            pltpu.VMEM((C,), jnp.int32),
            pltpu.VMEM((ROWS_PER_W,), jnp.int32),
            pltpu.VMEM((CHUNK, HW), jnp.float32),
            pltpu.VMEM((CHUNK, HW), jnp.float32),
            pltpu.SemaphoreType.DMA,
            pltpu.SemaphoreType.DMA,
            pltpu.SemaphoreType.DMA,
            pltpu.SemaphoreType.DMA,
        ],
        compiler_params=pltpu.CompilerParams(use_tc_tiling_on_sc=False),
    )(x2, indices)


def kernel(x, logdet, indices):
    out = _shuffle(x.reshape(BC, HW), indices)
    return (out.reshape(B, C, H, W), logdet)


# speedup vs baseline: 2.4071x; 2.4071x over previous
import jax
import jax.numpy as jnp
from jax import lax
from jax.experimental import pallas as pl
from jax.experimental.pallas import tpu as pltpu
from jax.experimental.pallas import tpu_sc as plsc

B, C, H, W = 64, 768, 28, 28
HW = H * W
NC, NS, L = 2, 16, 16
NW = NC * NS
CT = C // 128
BT = B // 8
NBLK = HW * BT
BLKW = CT * 8 * 128
BPW = NBLK // NW
CH = 4
CHW = CH * BLKW
NCHUNK = BPW // CH
NGRP = BLKW // L


def _sc_shuffle(x_hbm, idx_hbm, out_hbm, perm_v, sidx_v, table_v,
                in0, in1, out0, out1, gsem0, gsem1, ssem0, ssem1):
    cid = lax.axis_index("c")
    sid = lax.axis_index("s")
    wid = sid * NC + cid
    w_base = wid * BPW * BLKW

    lanes = lax.iota(jnp.int32, L)

    pltpu.sync_copy(idx_hbm, perm_v)
    for j in range(C // L):
        p = perm_v[pl.ds(j * L, L)]
        sidx_v[pl.ds(j * L, L)] = (p >> 7) * 1024 + (p & 127)

    def tbody(j, _):
        c1 = j // 64
        r = j % 64
        b2 = r // 8
        g = r % 8
        s16 = plsc.load_gather(sidx_v, [lanes + (c1 * 128 + g * L)])
        widx = lanes + (c1 * 1024 + b2 * 128 + g * L)
        plsc.store_scatter(table_v, [widx], s16 + b2 * 128)
        return 0

    lax.fori_loop(0, NGRP, tbody, 0)

    ins = (in0, in1)
    outs = (out0, out1)
    gsems = (gsem0, gsem1)
    ssems = (ssem0, ssem1)

    def gather(k):
        src = x_hbm.at[pl.ds(w_base + k * CHW, CHW)]
        return pltpu.async_copy(src, ins[k % 2], gsems[k % 2])

    def scatter(k):
        dst = out_hbm.at[pl.ds(w_base + k * CHW, CHW)]
        return pltpu.async_copy(outs[k % 2], dst, ssems[k % 2])

    def permute(k):
        src = ins[k % 2]
        dst = outs[k % 2]

        def body(j, _):
            widx = lanes + j * L
            srcw = plsc.load_gather(table_v, [widx])
            for b in range(CH):
                val = plsc.load_gather(src, [srcw + b * BLKW])
                plsc.store_scatter(dst, [widx + b * BLKW], val)
            return 0

        lax.fori_loop(0, NGRP, body, 0)

    g_descs = [None] * NCHUNK
    s_descs = [None] * NCHUNK
    g_descs[0] = gather(0)
    for k in range(NCHUNK):
        g_descs[k].wait()
        if k + 1 < NCHUNK:
            g_descs[k + 1] = gather(k + 1)
        if k >= 2:
            s_descs[k - 2].wait()
        permute(k)
        s_descs[k] = scatter(k)
    s_descs[NCHUNK - 2].wait()
    s_descs[NCHUNK - 1].wait()


@jax.jit
def _shuffle(x_flat, indices):
    mesh = plsc.VectorSubcoreMesh(core_axis_name="c", subcore_axis_name="s",
                                  num_cores=NC, num_subcores=NS)
    return pl.kernel(
        _sc_shuffle,
        out_type=jax.ShapeDtypeStruct((NBLK * BLKW,), jnp.float32),
        mesh=mesh,
        scratch_types=[
            pltpu.VMEM((C,), jnp.int32),
            pltpu.VMEM((C,), jnp.int32),
            pltpu.VMEM((BLKW,), jnp.int32),
            pltpu.VMEM((CHW,), jnp.float32),
            pltpu.VMEM((CHW,), jnp.float32),
            pltpu.VMEM((CHW,), jnp.float32),
            pltpu.VMEM((CHW,), jnp.float32),
            pltpu.SemaphoreType.DMA,
            pltpu.SemaphoreType.DMA,
            pltpu.SemaphoreType.DMA,
            pltpu.SemaphoreType.DMA,
        ],
        compiler_params=pltpu.CompilerParams(use_tc_tiling_on_sc=False,
                                             needs_layout_passes=False),
    )(x_flat, indices)


def kernel(x, logdet, indices):
    xf = (x.reshape(B, C, HW)
           .transpose(2, 0, 1)
           .reshape(HW, BT, 8, CT, 128)
           .transpose(0, 1, 3, 2, 4)
           .reshape(NBLK * BLKW))
    of = _shuffle(xf, indices)
    out = (of.reshape(HW, BT, CT, 8, 128)
             .transpose(0, 1, 3, 2, 4)
             .reshape(HW, B, C)
             .transpose(1, 2, 0)
             .reshape(B, C, H, W))
    return (out, logdet)


# speedup vs baseline: 7.1960x; 2.9895x over previous
import jax
import jax.numpy as jnp
from jax import lax
from jax.experimental import pallas as pl
from jax.experimental.pallas import tpu as pltpu
from jax.experimental.pallas import tpu_sc as plsc

B, C, H, W = 64, 768, 28, 28
HW = H * W
NC, NS, L = 2, 16, 16
NW = NC * NS
CT = C // 128
BT = B // 8
NBLK = HW * BT
BLKW = CT * 8 * 128
BPW = NBLK // NW
CH = 4
CHW = CH * BLKW
NCHUNK = BPW // CH
NGRP = BLKW // L


def _sc_shuffle(x_hbm, idx_hbm, out_hbm, perm_v, sidx_v, table_v,
                in0, in1, out0, out1, gsem0, gsem1, ssem0, ssem1):
    cid = lax.axis_index("c")
    sid = lax.axis_index("s")
    wid = sid * NC + cid
    w_base = wid * BPW * BLKW

    lanes = lax.iota(jnp.int32, L)

    pltpu.sync_copy(idx_hbm, perm_v)
    for j in range(C // L):
        p = perm_v[pl.ds(j * L, L)]
        sidx_v[pl.ds(j * L, L)] = (p >> 7) * 1024 + (p & 127)

    def tbody(j, _):
        c1 = j // 64
        r = j % 64
        b2 = r // 8
        g = r % 8
        s16 = plsc.load_gather(sidx_v, [lanes + (c1 * 128 + g * L)])
        widx = lanes + (c1 * 1024 + b2 * 128 + g * L)
        plsc.store_scatter(table_v, [widx], s16 + b2 * 128)
        return 0

    lax.fori_loop(0, NGRP, tbody, 0)

    ins = (in0, in1)
    outs = (out0, out1)
    gsems = (gsem0, gsem1)
    ssems = (ssem0, ssem1)

    def gather(k):
        src = x_hbm.at[pl.ds(w_base + k * CHW, CHW)]
        return pltpu.async_copy(src, ins[k % 2], gsems[k % 2])

    def scatter(k):
        dst = out_hbm.at[pl.ds(w_base + k * CHW, CHW)]
        return pltpu.async_copy(outs[k % 2], dst, ssems[k % 2])

    def permute(k):
        src = ins[k % 2]
        dst = outs[k % 2]

        @plsc.parallel_loop(0, NGRP, unroll=4)
        def _(j):
            widx = lanes + j * L
            srcw = plsc.load_gather(table_v, [widx])
            for b in range(CH):
                val = plsc.load_gather(src, [srcw + b * BLKW])
                plsc.store_scatter(dst, [widx + b * BLKW], val)

    g_descs = [None] * NCHUNK
    s_descs = [None] * NCHUNK
    g_descs[0] = gather(0)
    for k in range(NCHUNK):
        g_descs[k].wait()
        if k + 1 < NCHUNK:
            g_descs[k + 1] = gather(k + 1)
        if k >= 2:
            s_descs[k - 2].wait()
        permute(k)
        s_descs[k] = scatter(k)
    s_descs[NCHUNK - 2].wait()
    s_descs[NCHUNK - 1].wait()


@jax.jit
def _shuffle(x_flat, indices):
    mesh = plsc.VectorSubcoreMesh(core_axis_name="c", subcore_axis_name="s",
                                  num_cores=NC, num_subcores=NS)
    return pl.kernel(
        _sc_shuffle,
        out_type=jax.ShapeDtypeStruct((NBLK * BLKW,), jnp.float32),
        mesh=mesh,
        scratch_types=[
            pltpu.VMEM((C,), jnp.int32),
            pltpu.VMEM((C,), jnp.int32),
            pltpu.VMEM((BLKW,), jnp.int32),
            pltpu.VMEM((CHW,), jnp.float32),
            pltpu.VMEM((CHW,), jnp.float32),
            pltpu.VMEM((CHW,), jnp.float32),
            pltpu.VMEM((CHW,), jnp.float32),
            pltpu.SemaphoreType.DMA,
            pltpu.SemaphoreType.DMA,
            pltpu.SemaphoreType.DMA,
            pltpu.SemaphoreType.DMA,
        ],
        compiler_params=pltpu.CompilerParams(use_tc_tiling_on_sc=False,
                                             needs_layout_passes=False),
    )(x_flat, indices)


def kernel(x, logdet, indices):
    xf = (x.reshape(B, C, HW)
           .transpose(2, 0, 1)
           .reshape(HW, BT, 8, CT, 128)
           .transpose(0, 1, 3, 2, 4)
           .reshape(NBLK * BLKW))
    of = _shuffle(xf, indices)
    out = (of.reshape(HW, BT, CT, 8, 128)
             .transpose(0, 1, 3, 2, 4)
             .reshape(HW, B, C)
             .transpose(1, 2, 0)
             .reshape(B, C, H, W))
    return (out, logdet)


# speedup vs baseline: 7.4457x; 1.0347x over previous
import jax
import jax.numpy as jnp
from jax import lax
from jax.experimental import pallas as pl
from jax.experimental.pallas import tpu as pltpu
from jax.experimental.pallas import tpu_sc as plsc

B, C, H, W = 64, 768, 28, 28
HW = H * W
NC, NS, L = 2, 16, 16
NW = NC * NS
CT = C // 128
BT = B // 8
NBLK = HW * BT
BLKW = CT * 8 * 128
BPW = NBLK // NW
CH = 4
CHW = CH * BLKW
NCHUNK = BPW // CH
NGRP = BLKW // L


def _sc_shuffle(x_hbm, idx_hbm, out_hbm, perm_v, sidx_v, table_v,
                in0, in1, in2, out0, out1, gsem0, gsem1, gsem2, ssem0, ssem1):
    cid = lax.axis_index("c")
    sid = lax.axis_index("s")
    wid = sid * NC + cid
    w_base = wid * BPW * BLKW

    lanes = lax.iota(jnp.int32, L)

    pltpu.sync_copy(idx_hbm, perm_v)
    for j in range(C // L):
        p = perm_v[pl.ds(j * L, L)]
        sidx_v[pl.ds(j * L, L)] = (p >> 7) * 1024 + (p & 127)

    def tbody(j, _):
        c1 = j // 64
        r = j % 64
        b2 = r // 8
        g = r % 8
        s16 = plsc.load_gather(sidx_v, [lanes + (c1 * 128 + g * L)])
        widx = lanes + (c1 * 1024 + b2 * 128 + g * L)
        plsc.store_scatter(table_v, [widx], s16 + b2 * 128)
        return 0

    lax.fori_loop(0, NGRP, tbody, 0)

    ins = (in0, in1, in2)
    outs = (out0, out1)
    gsems = (gsem0, gsem1, gsem2)
    ssems = (ssem0, ssem1)

    def gather(k):
        src = x_hbm.at[pl.ds(w_base + k * CHW, CHW)]
        return pltpu.async_copy(src, ins[k % 3], gsems[k % 3])

    def scatter(k):
        dst = out_hbm.at[pl.ds(w_base + k * CHW, CHW)]
        return pltpu.async_copy(outs[k % 2], dst, ssems[k % 2])

    def permute(k):
        src = ins[k % 3]
        dst = outs[k % 2]

        @plsc.parallel_loop(0, NGRP, unroll=4)
        def _(j):
            widx = lanes + j * L
            srcw = plsc.load_gather(table_v, [widx])
            for b in range(CH):
                val = plsc.load_gather(src, [srcw + b * BLKW])
                plsc.store_scatter(dst, [widx + b * BLKW], val)

    g_descs = [None] * NCHUNK
    s_descs = [None] * NCHUNK
    g_descs[0] = gather(0)
    g_descs[1] = gather(1)
    for k in range(NCHUNK):
        g_descs[k].wait()
        if k + 2 < NCHUNK:
            g_descs[k + 2] = gather(k + 2)
        if k >= 2:
            s_descs[k - 2].wait()
        permute(k)
        s_descs[k] = scatter(k)
    s_descs[NCHUNK - 2].wait()
    s_descs[NCHUNK - 1].wait()


@jax.jit
def _shuffle(x_flat, indices):
    mesh = plsc.VectorSubcoreMesh(core_axis_name="c", subcore_axis_name="s",
                                  num_cores=NC, num_subcores=NS)
    return pl.kernel(
        _sc_shuffle,
        out_type=jax.ShapeDtypeStruct((NBLK * BLKW,), jnp.float32),
        mesh=mesh,
        scratch_types=[
            pltpu.VMEM((C,), jnp.int32),
            pltpu.VMEM((C,), jnp.int32),
            pltpu.VMEM((BLKW,), jnp.int32),
            pltpu.VMEM((CHW,), jnp.float32),
            pltpu.VMEM((CHW,), jnp.float32),
            pltpu.VMEM((CHW,), jnp.float32),
            pltpu.VMEM((CHW,), jnp.float32),
            pltpu.VMEM((CHW,), jnp.float32),
            pltpu.SemaphoreType.DMA,
            pltpu.SemaphoreType.DMA,
            pltpu.SemaphoreType.DMA,
            pltpu.SemaphoreType.DMA,
            pltpu.SemaphoreType.DMA,
        ],
        compiler_params=pltpu.CompilerParams(use_tc_tiling_on_sc=False,
                                             needs_layout_passes=False),
    )(x_flat, indices)


def kernel(x, logdet, indices):
    xf = (x.reshape(B, C, HW)
           .transpose(2, 0, 1)
           .reshape(HW, BT, 8, CT, 128)
           .transpose(0, 1, 3, 2, 4)
           .reshape(NBLK * BLKW))
    of = _shuffle(xf, indices)
    out = (of.reshape(HW, BT, CT, 8, 128)
             .transpose(0, 1, 3, 2, 4)
             .reshape(HW, B, C)
             .transpose(1, 2, 0)
             .reshape(B, C, H, W))
    return (out, logdet)
